# baseline (speedup 1.0000x reference)
import functools
import math

import jax
import jax.numpy as jnp
from jax import lax
from jax.experimental import pallas as pl
from jax.experimental.pallas import tpu as pltpu
from jax.experimental.pallas import tpu_sc as plsc

N = 10000
E = 320000
DF = 128
H = 256
D = 128
NS = 16
NC = 2
CH = 128
BLK = 16
NBLK = 10
CHUNKS = BLK * NBLK
E_PAD = CHUNKS * CH * NS - E
ACC_ROWS = 10112
BNEPS = 1e-5
NEG_HUGE = -3.0e38

@functools.cache
def _sc_mesh():
    return plsc.VectorSubcoreMesh(core_axis_name="c", subcore_axis_name="s",
                                  num_cores=NC, num_subcores=NS)


_ZERO_SL = ACC_ROWS // NS
_OUT_SL = 632
_OUT_SL_LAST = N - 15 * _OUT_SL


def _seg_sum_kernel(xlo_hbm, xhi_hbm, src_hbm, dst_hbm, z_hbm,
                    alo_hbm, ahi_hbm, src_v, dst_v, rows_a, rows_b, acc,
                    sem_a, sem_b):
    c = lax.axis_index("c")
    s = lax.axis_index("s")
    pltpu.sync_copy(z_hbm.at[pl.ds(s * _ZERO_SL, _ZERO_SL)],
                    acc.at[pl.ds(s * _ZERO_SL, _ZERO_SL)])
    plsc.subcore_barrier()

    def run(x_hbm):
        dummy = x_hbm.at[pl.ds(0, CH)]

        def blk_body(b, carry):
            pltpu.sync_copy(src_hbm.at[s, pl.ds(b * BLK, BLK)], src_v)
            pltpu.sync_copy(dst_hbm.at[s, pl.ds(b * BLK, BLK)], dst_v)
            pltpu.async_copy(x_hbm.at[src_v.at[0]], rows_a, sem_a)

            def pair_body(t, inner):
                pltpu.make_async_copy(dummy, rows_a, sem_a).wait()
                pltpu.async_copy(x_hbm.at[src_v.at[2 * t + 1]], rows_b, sem_b)
                pltpu.sync_copy(rows_a, acc.at[dst_v.at[2 * t]], add=True)
                pltpu.make_async_copy(dummy, rows_b, sem_b).wait()

                @pl.when(t < BLK // 2 - 1)
                def _():
                    pltpu.async_copy(x_hbm.at[src_v.at[2 * t + 2]], rows_a,
                                     sem_a)

                pltpu.sync_copy(rows_b, acc.at[dst_v.at[2 * t + 1]], add=True)
                return inner

            return lax.fori_loop(0, BLK // 2, pair_body, carry)

        lax.fori_loop(0, NBLK, blk_body, jnp.int32(0))

    @pl.when(c == 0)
    def _():
        run(xlo_hbm)

    @pl.when(c == 1)
    def _():
        run(xhi_hbm)

    plsc.subcore_barrier()

    def writeout(a_hbm):
        @pl.when(s < 15)
        def _():
            pltpu.sync_copy(acc.at[pl.ds(s * _OUT_SL, _OUT_SL)],
                            a_hbm.at[pl.ds(s * _OUT_SL, _OUT_SL)])

        @pl.when(s == 15)
        def _():
            pltpu.sync_copy(acc.at[pl.ds(15 * _OUT_SL, _OUT_SL_LAST)],
                            a_hbm.at[pl.ds(15 * _OUT_SL, _OUT_SL_LAST)])

    @pl.when(c == 0)
    def _():
        writeout(alo_hbm)

    @pl.when(c == 1)
    def _():
        writeout(ahi_hbm)


def _seg_sum_sc(xlo, xhi, srcp, dstp, zeros):
    return pl.kernel(
        _seg_sum_kernel,
        out_type=(jax.ShapeDtypeStruct((N, D), jnp.float32),
                  jax.ShapeDtypeStruct((N, D), jnp.float32)),
        mesh=_sc_mesh(),
        scratch_types=[
            pltpu.VMEM((BLK, CH), jnp.int32),
            pltpu.VMEM((BLK, CH), jnp.int32),
            pltpu.VMEM((CH, D), jnp.float32),
            pltpu.VMEM((CH, D), jnp.float32),
            pltpu.VMEM_SHARED((ACC_ROWS, D), jnp.float32),
            pltpu.SemaphoreType.DMA,
            pltpu.SemaphoreType.DMA,
        ],
    )(xlo, xhi, srcp, dstp, zeros)


FBLK = 8
FCAP_CH = 176
FCAP = FCAP_CH * CH
_EPB = FBLK * CH


def _edge_filter_kernel(src_hbm, dst_hbm, mask_hbm,
                        fsrc_hbm, fdst_hbm, cnt_hbm,
                        src_v, dst_v, mask_v, osrc_v, odst_v, cnt_v):
    c = lax.axis_index("c")
    s = lax.axis_index("s")

    @pl.when(c == 0)
    def _():
        pltpu.sync_copy(mask_hbm, mask_v)
        idx16 = lax.iota(jnp.int32, 16)

        def blk(b, off):
            pltpu.sync_copy(src_hbm.at[s, pl.ds(b * BLK * CH, BLK * CH)],
                            src_v)
            pltpu.sync_copy(dst_hbm.at[s, pl.ds(b * BLK * CH, BLK * CH)],
                            dst_v)

            def grp(g, off2):
                o = pl.multiple_of(g * 16, 16)
                sv = src_v[pl.ds(o, 16)]
                dv = dst_v[pl.ds(o, 16)]
                ms = plsc.load_gather(mask_v, [sv])
                md = plsc.load_gather(mask_v, [dv])
                ok = (ms > 0.0) & (md > 0.0)
                pos = plsc.cumsum(ok.astype(jnp.int32))
                tgt = off2 + pos - 1
                plsc.store_scatter(osrc_v, [tgt], sv, mask=ok)
                plsc.store_scatter(odst_v, [tgt], dv, mask=ok)
                return off2 + jnp.max(pos)

            return lax.fori_loop(0, BLK * CH // 16, grp, off)

        off = lax.fori_loop(0, NBLK, blk, jnp.int32(0))

        zero16 = jnp.zeros((16,), jnp.int32)
        dump16 = N + ((idx16 + s * 16) & 63)

        def fill(t, carry):
            tgt = off + t * 16 + idx16
            plsc.store_scatter(osrc_v, [tgt], zero16)
            plsc.store_scatter(odst_v, [tgt], dump16)
            return carry

        lax.fori_loop(0, _EPB // 16, fill, jnp.int32(0))
        nblk = (off + _EPB - 1) // _EPB
        cnt_v[...] = jnp.full((16,), nblk, jnp.int32)
        pltpu.sync_copy(osrc_v, fsrc_hbm.at[s])
        pltpu.sync_copy(odst_v, fdst_hbm.at[s])
        pltpu.sync_copy(cnt_v, cnt_hbm.at[s])


def _edge_filter(src2d, dst2d, mask1d):
    return pl.kernel(
        _edge_filter_kernel,
        out_type=(jax.ShapeDtypeStruct((NS, FCAP), jnp.int32),
                  jax.ShapeDtypeStruct((NS, FCAP), jnp.int32),
                  jax.ShapeDtypeStruct((NS, 16), jnp.int32)),
        mesh=_sc_mesh(),
        compiler_params=pltpu.CompilerParams(needs_layout_passes=False),
        scratch_types=[
            pltpu.VMEM((BLK * CH,), jnp.int32),
            pltpu.VMEM((BLK * CH,), jnp.int32),
            pltpu.VMEM((N,), jnp.float32),
            pltpu.VMEM((FCAP,), jnp.int32),
            pltpu.VMEM((FCAP,), jnp.int32),
            pltpu.VMEM((16,), jnp.int32),
        ],
    )(src2d, dst2d, mask1d)


def _seg_sum_f_kernel(xlo_hbm, xhi_hbm, fsrc_hbm, fdst_hbm, cnt_hbm, z_hbm,
                      alo_hbm, ahi_hbm,
                      src_v, dst_v, rows_a, rows_b, cnt_v, acc, sem_a, sem_b):
    c = lax.axis_index("c")
    s = lax.axis_index("s")
    pltpu.sync_copy(z_hbm.at[pl.ds(s * _ZERO_SL, _ZERO_SL)],
                    acc.at[pl.ds(s * _ZERO_SL, _ZERO_SL)])
    pltpu.sync_copy(cnt_hbm.at[s], cnt_v)
    plsc.subcore_barrier()
    nblk = cnt_v[...][0]

    def run(x_hbm):
        dummy = x_hbm.at[pl.ds(0, CH)]

        def blk_body(b, carry):
            pltpu.sync_copy(fsrc_hbm.at[s, pl.ds(b * FBLK, FBLK)], src_v)
            pltpu.sync_copy(fdst_hbm.at[s, pl.ds(b * FBLK, FBLK)], dst_v)
            pltpu.async_copy(x_hbm.at[src_v.at[0]], rows_a, sem_a)

            def pair_body(t, inner):
                pltpu.make_async_copy(dummy, rows_a, sem_a).wait()
                pltpu.async_copy(x_hbm.at[src_v.at[2 * t + 1]], rows_b, sem_b)
                pltpu.sync_copy(rows_a, acc.at[dst_v.at[2 * t]], add=True)
                pltpu.make_async_copy(dummy, rows_b, sem_b).wait()

                @pl.when(t < FBLK // 2 - 1)
                def _():
                    pltpu.async_copy(x_hbm.at[src_v.at[2 * t + 2]], rows_a,
                                     sem_a)

                pltpu.sync_copy(rows_b, acc.at[dst_v.at[2 * t + 1]], add=True)
                return inner

            return lax.fori_loop(0, FBLK // 2, pair_body, carry)

        lax.fori_loop(0, nblk, blk_body, jnp.int32(0))

    @pl.when(c == 0)
    def _():
        run(xlo_hbm)

    @pl.when(c == 1)
    def _():
        run(xhi_hbm)

    plsc.subcore_barrier()

    def writeout(a_hbm):
        @pl.when(s < 15)
        def _():
            pltpu.sync_copy(acc.at[pl.ds(s * _OUT_SL, _OUT_SL)],
                            a_hbm.at[pl.ds(s * _OUT_SL, _OUT_SL)])

        @pl.when(s == 15)
        def _():
            pltpu.sync_copy(acc.at[pl.ds(15 * _OUT_SL, _OUT_SL_LAST)],
                            a_hbm.at[pl.ds(15 * _OUT_SL, _OUT_SL_LAST)])

    @pl.when(c == 0)
    def _():
        writeout(alo_hbm)

    @pl.when(c == 1)
    def _():
        writeout(ahi_hbm)


def _seg_sum_f_sc(xlo, xhi, fsrc, fdst, cnts, zeros):
    return pl.kernel(
        _seg_sum_f_kernel,
        out_type=(jax.ShapeDtypeStruct((N, D), jnp.float32),
                  jax.ShapeDtypeStruct((N, D), jnp.float32)),
        mesh=_sc_mesh(),
        compiler_params=pltpu.CompilerParams(needs_layout_passes=False),
        scratch_types=[
            pltpu.VMEM((FBLK, CH), jnp.int32),
            pltpu.VMEM((FBLK, CH), jnp.int32),
            pltpu.VMEM((CH, D), jnp.float32),
            pltpu.VMEM((CH, D), jnp.float32),
            pltpu.VMEM((16,), jnp.int32),
            pltpu.VMEM_SHARED((ACC_ROWS, D), jnp.float32),
            pltpu.SemaphoreType.DMA,
            pltpu.SemaphoreType.DMA,
        ],
    )(xlo, xhi, fsrc.reshape(NS, FCAP_CH, CH), fdst.reshape(NS, FCAP_CH, CH),
      cnts, zeros)


def _emb_kernel(x_ref, w_ref, b_ref, lo_ref, hi_ref):
    h = jnp.dot(x_ref[...], w_ref[...], preferred_element_type=jnp.float32)
    h = jnp.maximum(h + b_ref[...], 0.0)
    lo_ref[...] = h[:, :D]
    hi_ref[...] = h[:, D:]


def _emb(x, w, b):
    return pl.pallas_call(
        _emb_kernel,
        out_shape=(jax.ShapeDtypeStruct((N, D), jnp.float32),
                   jax.ShapeDtypeStruct((N, D), jnp.float32)),
    )(x, w, b)


def _masked_bn(h, m, k, g, b):
    mu = jnp.sum(h * m, axis=0, keepdims=True) * (1.0 / k)
    d = h - mu
    var = jnp.sum(d * d * m, axis=0, keepdims=True) * (1.0 / k)
    return d * lax.rsqrt(var + BNEPS) * g + b


def _layer_a_kernel(k_prev,
                    xlo_ref, xhi_ref, alo_ref, ahi_ref, m_ref,
                    w1_ref, b1_ref, g1_ref, bb1_ref,
                    hlo_ref, hhi_ref):
    m = m_ref[...]
    h = (jnp.dot(xlo_ref[...] + alo_ref[...], w1_ref[:D, :],
                 preferred_element_type=jnp.float32)
         + jnp.dot(xhi_ref[...] + ahi_ref[...], w1_ref[D:, :],
                   preferred_element_type=jnp.float32)
         + b1_ref[...])
    h = jnp.maximum(_masked_bn(h, m, k_prev, g1_ref[...], bb1_ref[...]), 0.0)
    hlo_ref[...] = h[:, :D]
    hhi_ref[...] = h[:, D:]


def _layer_b_kernel(k_prev, k_new,
                    hlo_ref, hhi_ref, m_ref,
                    w2_ref, b2_ref, g2_ref, bb2_ref, pw_ref,
                    xnlo_ref, xnhi_ref, mn_ref, read_ref):
    m = m_ref[...]
    h = (jnp.dot(hlo_ref[...], w2_ref[:D, :],
                 preferred_element_type=jnp.float32)
         + jnp.dot(hhi_ref[...], w2_ref[D:, :],
                   preferred_element_type=jnp.float32)
         + b2_ref[...])
    y = _masked_bn(h, m, k_prev, g2_ref[...], bb2_ref[...])
    y = jnp.where(y > 0, y, 0.1 * y)

    pw = pw_ref[...]
    nrm = jnp.sqrt(jnp.sum(pw * pw)) + 1e-16
    score = jnp.tanh(jnp.dot(y, pw, preferred_element_type=jnp.float32) / nrm)
    sm = jnp.where(m > 0, score, -2.0)

    u = lax.bitcast_convert_type(sm, jnp.uint32)
    ukey = jnp.where((u >> 31) != 0, ~u, u | jnp.uint32(0x80000000))

    def t_body(i, prefix):
        cand = prefix | (jnp.uint32(1) << (31 - i).astype(jnp.uint32))
        cnt = jnp.sum((ukey >= cand).astype(jnp.int32))
        return jnp.where(cnt >= k_new, cand, prefix)

    tkey = lax.fori_loop(0, 32, t_body, jnp.uint32(0))
    c_gt = jnp.sum((ukey > tkey).astype(jnp.int32))
    mrem = k_new - c_gt

    def r_body(i, prefix):
        bit = jnp.int32(1) << (13 - i).astype(jnp.int32)
        cap = prefix | (bit - 1)
        idx = lax.broadcasted_iota(jnp.int32, (N, 1), 0)
        cnt = jnp.sum(((ukey == tkey) & (idx <= cap)).astype(jnp.int32))
        return jnp.where(cnt >= mrem, prefix, prefix | bit)

    ridx = lax.fori_loop(0, 14, r_body, jnp.int32(0))
    idx = lax.broadcasted_iota(jnp.int32, (N, 1), 0)
    sel = (ukey > tkey) | ((ukey == tkey) & (idx <= ridx) & (mrem > 0))
    mn = sel.astype(jnp.float32)
    mn_ref[...] = mn
    sc = score * mn
    xnlo = y[:, :D] * sc
    xnhi = y[:, D:] * sc
    xnlo_ref[...] = xnlo
    xnhi_ref[...] = xnhi
    mxlo = jnp.max(jnp.where(sel, xnlo, NEG_HUGE), axis=0, keepdims=True)
    mxhi = jnp.max(jnp.where(sel, xnhi, NEG_HUGE), axis=0, keepdims=True)
    mnlo = jnp.sum(xnlo, axis=0, keepdims=True) * (1.0 / k_new)
    mnhi = jnp.sum(xnhi, axis=0, keepdims=True) * (1.0 / k_new)
    read_ref[...] = jnp.concatenate([mxlo, mxhi, mnlo, mnhi], axis=1)


def _layer(k_prev, k_new, xlo, xhi, alo, ahi, mask, cp, g, b, pw):
    hlo, hhi = pl.pallas_call(
        functools.partial(_layer_a_kernel, k_prev),
        out_shape=(jax.ShapeDtypeStruct((N, D), jnp.float32),
                   jax.ShapeDtypeStruct((N, D), jnp.float32)),
    )(xlo, xhi, alo, ahi, mask,
      cp['lin1_W'], cp['lin1_b'].reshape(1, H), cp['bn_g'].reshape(1, H),
      cp['bn_b'].reshape(1, H))
    return pl.pallas_call(
        functools.partial(_layer_b_kernel, k_prev, k_new),
        out_shape=(jax.ShapeDtypeStruct((N, D), jnp.float32),
                   jax.ShapeDtypeStruct((N, D), jnp.float32),
                   jax.ShapeDtypeStruct((N, 1), jnp.float32),
                   jax.ShapeDtypeStruct((1, 2 * H), jnp.float32)),
        compiler_params=pltpu.CompilerParams(
            vmem_limit_bytes=64 * 1024 * 1024),
    )(hlo, hhi, mask,
      cp['lin2_W'], cp['lin2_b'].reshape(1, H),
      g.reshape(1, H), b.reshape(1, H), pw.reshape(H, 1))


def _final_kernel(r1_ref, r2_ref, r3_ref, w_ref, b_ref, out_ref):
    def lk(v):
        return jnp.where(v > 0, v, 0.1 * v)

    agg = lk(r1_ref[...]) + lk(r2_ref[...]) + lk(r3_ref[...])
    out_ref[...] = (jnp.dot(agg, w_ref[...], preferred_element_type=jnp.float32)
                    + b_ref[...])


def _final(r1, r2, r3, w, b):
    return pl.pallas_call(
        _final_kernel,
        out_shape=jax.ShapeDtypeStruct((1, H), jnp.float32),
    )(r1, r2, r3, w, b.reshape(1, H))


def kernel(x, edge_index, batch, params):
    p = params
    src = edge_index[0].astype(jnp.int32)
    dst = edge_index[1].astype(jnp.int32)
    src2d = jnp.concatenate([src, jnp.zeros((E_PAD,), jnp.int32)]
                            ).reshape(NS, CHUNKS * CH)
    dst2d = jnp.concatenate(
        [dst, N + (jnp.arange(E_PAD, dtype=jnp.int32) % 64)]
    ).reshape(NS, CHUNKS * CH)
    srcp = src2d.reshape(NS, CHUNKS, CH)
    dstp = dst2d.reshape(NS, CHUNKS, CH)
    zeros = jnp.zeros((ACC_ROWS, D), jnp.float32)

    xlo, xhi = _emb(x, p['emb_W'], p['emb_b'].reshape(1, H))
    mask = jnp.ones((N, 1), jnp.float32)
    k = N
    reads = []
    for i in (1, 2, 3):
        if i == 1:
            alo, ahi = _seg_sum_sc(xlo, xhi, srcp, dstp, zeros)
        else:
            fsrc, fdst, cnts = _edge_filter(src2d, dst2d, mask.reshape(N))
            alo, ahi = _seg_sum_f_sc(xlo, xhi, fsrc, fdst, cnts, zeros)
        k_new = int(math.ceil(0.5 * k))
        xlo, xhi, mask, read = _layer(
            k, k_new, xlo, xhi, alo, ahi, mask, p['conv%d' % i],
            p['bn%d_g' % i], p['bn%d_b' % i], p['pool%d_w' % i])
        k = k_new
        reads.append(read)
    return _final(reads[0], reads[1], reads[2], p['lin1_W'], p['lin1_b'])


# speedup vs baseline: 2.0420x; 2.0420x over previous
import functools
import math

import jax
import jax.numpy as jnp
from jax import lax
from jax.experimental import pallas as pl
from jax.experimental.pallas import tpu as pltpu
from jax.experimental.pallas import tpu_sc as plsc

N = 10000
E = 320000
DF = 128
H = 256
D = 128
NS = 16
NC = 2
CH = 128
BLK = 16
NBLK = 10
CHUNKS = BLK * NBLK
E_PAD = CHUNKS * CH * NS - E
ACC_ROWS = 10112
BNEPS = 1e-5
NEG_HUGE = -3.0e38

@functools.cache
def _sc_mesh():
    return plsc.VectorSubcoreMesh(core_axis_name="c", subcore_axis_name="s",
                                  num_cores=NC, num_subcores=NS)


_ZERO_SL = ACC_ROWS // NS
_OUT_SL = 632
_OUT_SL_LAST = N - 15 * _OUT_SL


def _seg_sum_kernel(xlo_hbm, xhi_hbm, src_hbm, dst_hbm, z_hbm,
                    alo_hbm, ahi_hbm, src_v, dst_v, rows_a, rows_b, acc,
                    sem_a, sem_b):
    c = lax.axis_index("c")
    s = lax.axis_index("s")
    pltpu.sync_copy(z_hbm.at[pl.ds(s * _ZERO_SL, _ZERO_SL)],
                    acc.at[pl.ds(s * _ZERO_SL, _ZERO_SL)])
    plsc.subcore_barrier()

    def run(x_hbm):
        dummy = x_hbm.at[pl.ds(0, CH)]

        def blk_body(b, carry):
            pltpu.sync_copy(src_hbm.at[s, pl.ds(b * BLK, BLK)], src_v)
            pltpu.sync_copy(dst_hbm.at[s, pl.ds(b * BLK, BLK)], dst_v)
            pltpu.async_copy(x_hbm.at[src_v.at[0]], rows_a, sem_a)

            def pair_body(t, inner):
                pltpu.make_async_copy(dummy, rows_a, sem_a).wait()
                pltpu.async_copy(x_hbm.at[src_v.at[2 * t + 1]], rows_b, sem_b)
                pltpu.sync_copy(rows_a, acc.at[dst_v.at[2 * t]], add=True)
                pltpu.make_async_copy(dummy, rows_b, sem_b).wait()

                @pl.when(t < BLK // 2 - 1)
                def _():
                    pltpu.async_copy(x_hbm.at[src_v.at[2 * t + 2]], rows_a,
                                     sem_a)

                pltpu.sync_copy(rows_b, acc.at[dst_v.at[2 * t + 1]], add=True)
                return inner

            return lax.fori_loop(0, BLK // 2, pair_body, carry)

        lax.fori_loop(0, NBLK, blk_body, jnp.int32(0))

    @pl.when(c == 0)
    def _():
        run(xlo_hbm)

    @pl.when(c == 1)
    def _():
        run(xhi_hbm)

    plsc.subcore_barrier()

    def writeout(a_hbm):
        @pl.when(s < 15)
        def _():
            pltpu.sync_copy(acc.at[pl.ds(s * _OUT_SL, _OUT_SL)],
                            a_hbm.at[pl.ds(s * _OUT_SL, _OUT_SL)])

        @pl.when(s == 15)
        def _():
            pltpu.sync_copy(acc.at[pl.ds(15 * _OUT_SL, _OUT_SL_LAST)],
                            a_hbm.at[pl.ds(15 * _OUT_SL, _OUT_SL_LAST)])

    @pl.when(c == 0)
    def _():
        writeout(alo_hbm)

    @pl.when(c == 1)
    def _():
        writeout(ahi_hbm)


def _seg_sum_sc(xlo, xhi, srcp, dstp, zeros):
    return pl.kernel(
        _seg_sum_kernel,
        out_type=(jax.ShapeDtypeStruct((N, D), jnp.float32),
                  jax.ShapeDtypeStruct((N, D), jnp.float32)),
        mesh=_sc_mesh(),
        scratch_types=[
            pltpu.VMEM((BLK, CH), jnp.int32),
            pltpu.VMEM((BLK, CH), jnp.int32),
            pltpu.VMEM((CH, D), jnp.float32),
            pltpu.VMEM((CH, D), jnp.float32),
            pltpu.VMEM_SHARED((ACC_ROWS, D), jnp.float32),
            pltpu.SemaphoreType.DMA,
            pltpu.SemaphoreType.DMA,
        ],
    )(xlo, xhi, srcp, dstp, zeros)


FBLK = 8
FCAP_CH = 176
FCAP = FCAP_CH * CH
_EPB = FBLK * CH


def _edge_filter_kernel(src_hbm, dst_hbm, mask_hbm,
                        fsrc_hbm, fdst_hbm, cnt_hbm,
                        src_v, dst_v, mask_v, osrc_v, odst_v, cnt_v):
    c = lax.axis_index("c")
    s = lax.axis_index("s")

    @pl.when(c == 0)
    def _():
        pltpu.sync_copy(mask_hbm, mask_v)
        idx16 = lax.iota(jnp.int32, 16)

        def blk(b, off):
            pltpu.sync_copy(src_hbm.at[s, pl.ds(b * BLK * CH, BLK * CH)],
                            src_v)
            pltpu.sync_copy(dst_hbm.at[s, pl.ds(b * BLK * CH, BLK * CH)],
                            dst_v)

            def grp(g, off2):
                o = pl.multiple_of(g * 16, 16)
                sv = src_v[pl.ds(o, 16)]
                dv = dst_v[pl.ds(o, 16)]
                ms = plsc.load_gather(mask_v, [sv])
                md = plsc.load_gather(mask_v, [dv])
                ok = (ms > 0.0) & (md > 0.0)
                pos = plsc.cumsum(ok.astype(jnp.int32))
                tgt = off2 + pos - 1
                plsc.store_scatter(osrc_v, [tgt], sv, mask=ok)
                plsc.store_scatter(odst_v, [tgt], dv, mask=ok)
                return off2 + jnp.max(pos)

            return lax.fori_loop(0, BLK * CH // 16, grp, off)

        off = lax.fori_loop(0, NBLK, blk, jnp.int32(0))

        dump16 = N + ((idx16 + s * 16) & 63)

        def fill(t, carry):
            tgt = off + t * 16 + idx16
            srcf = (idx16 * 613 + t * 89 + s * 509) & 8191
            plsc.store_scatter(osrc_v, [tgt], srcf)
            plsc.store_scatter(odst_v, [tgt], dump16)
            return carry

        lax.fori_loop(0, _EPB // 16, fill, jnp.int32(0))
        nblk = (off + _EPB - 1) // _EPB
        cnt_v[...] = jnp.full((16,), nblk, jnp.int32)
        pltpu.sync_copy(osrc_v, fsrc_hbm.at[s])
        pltpu.sync_copy(odst_v, fdst_hbm.at[s])
        pltpu.sync_copy(cnt_v, cnt_hbm.at[s])


def _edge_filter(src2d, dst2d, mask1d):
    return pl.kernel(
        _edge_filter_kernel,
        out_type=(jax.ShapeDtypeStruct((NS, FCAP), jnp.int32),
                  jax.ShapeDtypeStruct((NS, FCAP), jnp.int32),
                  jax.ShapeDtypeStruct((NS, 16), jnp.int32)),
        mesh=_sc_mesh(),
        compiler_params=pltpu.CompilerParams(needs_layout_passes=False),
        scratch_types=[
            pltpu.VMEM((BLK * CH,), jnp.int32),
            pltpu.VMEM((BLK * CH,), jnp.int32),
            pltpu.VMEM((N,), jnp.float32),
            pltpu.VMEM((FCAP,), jnp.int32),
            pltpu.VMEM((FCAP,), jnp.int32),
            pltpu.VMEM((16,), jnp.int32),
        ],
    )(src2d, dst2d, mask1d)


def _seg_sum_f_kernel(xlo_hbm, xhi_hbm, fsrc_hbm, fdst_hbm, cnt_hbm, z_hbm,
                      alo_hbm, ahi_hbm,
                      src_v, dst_v, rows_a, rows_b, cnt_v, acc, sem_a, sem_b):
    c = lax.axis_index("c")
    s = lax.axis_index("s")
    pltpu.sync_copy(z_hbm.at[pl.ds(s * _ZERO_SL, _ZERO_SL)],
                    acc.at[pl.ds(s * _ZERO_SL, _ZERO_SL)])
    pltpu.sync_copy(cnt_hbm.at[s], cnt_v)
    plsc.subcore_barrier()
    nblk = cnt_v[...][0]

    def run(x_hbm):
        dummy = x_hbm.at[pl.ds(0, CH)]

        def blk_body(b, carry):
            pltpu.sync_copy(fsrc_hbm.at[s, pl.ds(b * FBLK, FBLK)], src_v)
            pltpu.sync_copy(fdst_hbm.at[s, pl.ds(b * FBLK, FBLK)], dst_v)
            pltpu.async_copy(x_hbm.at[src_v.at[0]], rows_a, sem_a)

            def pair_body(t, inner):
                pltpu.make_async_copy(dummy, rows_a, sem_a).wait()
                pltpu.async_copy(x_hbm.at[src_v.at[2 * t + 1]], rows_b, sem_b)
                pltpu.sync_copy(rows_a, acc.at[dst_v.at[2 * t]], add=True)
                pltpu.make_async_copy(dummy, rows_b, sem_b).wait()

                @pl.when(t < FBLK // 2 - 1)
                def _():
                    pltpu.async_copy(x_hbm.at[src_v.at[2 * t + 2]], rows_a,
                                     sem_a)

                pltpu.sync_copy(rows_b, acc.at[dst_v.at[2 * t + 1]], add=True)
                return inner

            return lax.fori_loop(0, FBLK // 2, pair_body, carry)

        lax.fori_loop(0, nblk, blk_body, jnp.int32(0))

    @pl.when(c == 0)
    def _():
        run(xlo_hbm)

    @pl.when(c == 1)
    def _():
        run(xhi_hbm)

    plsc.subcore_barrier()

    def writeout(a_hbm):
        @pl.when(s < 15)
        def _():
            pltpu.sync_copy(acc.at[pl.ds(s * _OUT_SL, _OUT_SL)],
                            a_hbm.at[pl.ds(s * _OUT_SL, _OUT_SL)])

        @pl.when(s == 15)
        def _():
            pltpu.sync_copy(acc.at[pl.ds(15 * _OUT_SL, _OUT_SL_LAST)],
                            a_hbm.at[pl.ds(15 * _OUT_SL, _OUT_SL_LAST)])

    @pl.when(c == 0)
    def _():
        writeout(alo_hbm)

    @pl.when(c == 1)
    def _():
        writeout(ahi_hbm)


def _seg_sum_f_sc(xlo, xhi, fsrc, fdst, cnts, zeros):
    return pl.kernel(
        _seg_sum_f_kernel,
        out_type=(jax.ShapeDtypeStruct((N, D), jnp.float32),
                  jax.ShapeDtypeStruct((N, D), jnp.float32)),
        mesh=_sc_mesh(),
        compiler_params=pltpu.CompilerParams(needs_layout_passes=False),
        scratch_types=[
            pltpu.VMEM((FBLK, CH), jnp.int32),
            pltpu.VMEM((FBLK, CH), jnp.int32),
            pltpu.VMEM((CH, D), jnp.float32),
            pltpu.VMEM((CH, D), jnp.float32),
            pltpu.VMEM((16,), jnp.int32),
            pltpu.VMEM_SHARED((ACC_ROWS, D), jnp.float32),
            pltpu.SemaphoreType.DMA,
            pltpu.SemaphoreType.DMA,
        ],
    )(xlo, xhi, fsrc.reshape(NS, FCAP_CH, CH), fdst.reshape(NS, FCAP_CH, CH),
      cnts, zeros)


def _emb_kernel(x_ref, w_ref, b_ref, lo_ref, hi_ref):
    h = jnp.dot(x_ref[...], w_ref[...], preferred_element_type=jnp.float32)
    h = jnp.maximum(h + b_ref[...], 0.0)
    lo_ref[...] = h[:, :D]
    hi_ref[...] = h[:, D:]


def _emb(x, w, b):
    return pl.pallas_call(
        _emb_kernel,
        out_shape=(jax.ShapeDtypeStruct((N, D), jnp.float32),
                   jax.ShapeDtypeStruct((N, D), jnp.float32)),
    )(x, w, b)


def _masked_bn(h, m, k, g, b):
    mu = jnp.sum(h * m, axis=0, keepdims=True) * (1.0 / k)
    d = h - mu
    var = jnp.sum(d * d * m, axis=0, keepdims=True) * (1.0 / k)
    return d * lax.rsqrt(var + BNEPS) * g + b


def _layer_a_kernel(k_prev,
                    xlo_ref, xhi_ref, alo_ref, ahi_ref, m_ref,
                    w1_ref, b1_ref, g1_ref, bb1_ref,
                    hlo_ref, hhi_ref):
    m = m_ref[...]
    h = (jnp.dot(xlo_ref[...] + alo_ref[...], w1_ref[:D, :],
                 preferred_element_type=jnp.float32)
         + jnp.dot(xhi_ref[...] + ahi_ref[...], w1_ref[D:, :],
                   preferred_element_type=jnp.float32)
         + b1_ref[...])
    h = jnp.maximum(_masked_bn(h, m, k_prev, g1_ref[...], bb1_ref[...]), 0.0)
    hlo_ref[...] = h[:, :D]
    hhi_ref[...] = h[:, D:]


def _layer_b_kernel(k_prev, k_new,
                    hlo_ref, hhi_ref, m_ref,
                    w2_ref, b2_ref, g2_ref, bb2_ref, pw_ref,
                    xnlo_ref, xnhi_ref, mn_ref, read_ref):
    m = m_ref[...]
    h = (jnp.dot(hlo_ref[...], w2_ref[:D, :],
                 preferred_element_type=jnp.float32)
         + jnp.dot(hhi_ref[...], w2_ref[D:, :],
                   preferred_element_type=jnp.float32)
         + b2_ref[...])
    y = _masked_bn(h, m, k_prev, g2_ref[...], bb2_ref[...])
    y = jnp.where(y > 0, y, 0.1 * y)

    pw = pw_ref[...]
    nrm = jnp.sqrt(jnp.sum(pw * pw)) + 1e-16
    score = jnp.tanh(jnp.dot(y, pw, preferred_element_type=jnp.float32) / nrm)
    sm = jnp.where(m > 0, score, -2.0)

    u = lax.bitcast_convert_type(sm, jnp.uint32)
    ukey = jnp.where((u >> 31) != 0, ~u, u | jnp.uint32(0x80000000))

    def t_body(i, prefix):
        cand = prefix | (jnp.uint32(1) << (31 - i).astype(jnp.uint32))
        cnt = jnp.sum((ukey >= cand).astype(jnp.int32))
        return jnp.where(cnt >= k_new, cand, prefix)

    tkey = lax.fori_loop(0, 32, t_body, jnp.uint32(0))
    c_gt = jnp.sum((ukey > tkey).astype(jnp.int32))
    mrem = k_new - c_gt

    def r_body(i, prefix):
        bit = jnp.int32(1) << (13 - i).astype(jnp.int32)
        cap = prefix | (bit - 1)
        idx = lax.broadcasted_iota(jnp.int32, (N, 1), 0)
        cnt = jnp.sum(((ukey == tkey) & (idx <= cap)).astype(jnp.int32))
        return jnp.where(cnt >= mrem, prefix, prefix | bit)

    ridx = lax.fori_loop(0, 14, r_body, jnp.int32(0))
    idx = lax.broadcasted_iota(jnp.int32, (N, 1), 0)
    sel = (ukey > tkey) | ((ukey == tkey) & (idx <= ridx) & (mrem > 0))
    mn = sel.astype(jnp.float32)
    mn_ref[...] = mn
    sc = score * mn
    xnlo = y[:, :D] * sc
    xnhi = y[:, D:] * sc
    xnlo_ref[...] = xnlo
    xnhi_ref[...] = xnhi
    mxlo = jnp.max(jnp.where(sel, xnlo, NEG_HUGE), axis=0, keepdims=True)
    mxhi = jnp.max(jnp.where(sel, xnhi, NEG_HUGE), axis=0, keepdims=True)
    mnlo = jnp.sum(xnlo, axis=0, keepdims=True) * (1.0 / k_new)
    mnhi = jnp.sum(xnhi, axis=0, keepdims=True) * (1.0 / k_new)
    read_ref[...] = jnp.concatenate([mxlo, mxhi, mnlo, mnhi], axis=1)


def _layer(k_prev, k_new, xlo, xhi, alo, ahi, mask, cp, g, b, pw):
    hlo, hhi = pl.pallas_call(
        functools.partial(_layer_a_kernel, k_prev),
        out_shape=(jax.ShapeDtypeStruct((N, D), jnp.float32),
                   jax.ShapeDtypeStruct((N, D), jnp.float32)),
    )(xlo, xhi, alo, ahi, mask,
      cp['lin1_W'], cp['lin1_b'].reshape(1, H), cp['bn_g'].reshape(1, H),
      cp['bn_b'].reshape(1, H))
    return pl.pallas_call(
        functools.partial(_layer_b_kernel, k_prev, k_new),
        out_shape=(jax.ShapeDtypeStruct((N, D), jnp.float32),
                   jax.ShapeDtypeStruct((N, D), jnp.float32),
                   jax.ShapeDtypeStruct((N, 1), jnp.float32),
                   jax.ShapeDtypeStruct((1, 2 * H), jnp.float32)),
        compiler_params=pltpu.CompilerParams(
            vmem_limit_bytes=64 * 1024 * 1024),
    )(hlo, hhi, mask,
      cp['lin2_W'], cp['lin2_b'].reshape(1, H),
      g.reshape(1, H), b.reshape(1, H), pw.reshape(H, 1))


def _final_kernel(r1_ref, r2_ref, r3_ref, w_ref, b_ref, out_ref):
    def lk(v):
        return jnp.where(v > 0, v, 0.1 * v)

    agg = lk(r1_ref[...]) + lk(r2_ref[...]) + lk(r3_ref[...])
    out_ref[...] = (jnp.dot(agg, w_ref[...], preferred_element_type=jnp.float32)
                    + b_ref[...])


def _final(r1, r2, r3, w, b):
    return pl.pallas_call(
        _final_kernel,
        out_shape=jax.ShapeDtypeStruct((1, H), jnp.float32),
    )(r1, r2, r3, w, b.reshape(1, H))


def kernel(x, edge_index, batch, params):
    p = params
    src = edge_index[0].astype(jnp.int32)
    dst = edge_index[1].astype(jnp.int32)
    src2d = jnp.concatenate(
        [src, (jnp.arange(E_PAD, dtype=jnp.int32) * 613) & 8191]
    ).reshape(NS, CHUNKS * CH)
    dst2d = jnp.concatenate(
        [dst, N + (jnp.arange(E_PAD, dtype=jnp.int32) % 64)]
    ).reshape(NS, CHUNKS * CH)
    srcp = src2d.reshape(NS, CHUNKS, CH)
    dstp = dst2d.reshape(NS, CHUNKS, CH)
    zeros = jnp.zeros((ACC_ROWS, D), jnp.float32)

    xlo, xhi = _emb(x, p['emb_W'], p['emb_b'].reshape(1, H))
    mask = jnp.ones((N, 1), jnp.float32)
    k = N
    reads = []
    for i in (1, 2, 3):
        if i == 1:
            alo, ahi = _seg_sum_sc(xlo, xhi, srcp, dstp, zeros)
        else:
            fsrc, fdst, cnts = _edge_filter(src2d, dst2d, mask.reshape(N))
            alo, ahi = _seg_sum_f_sc(xlo, xhi, fsrc, fdst, cnts, zeros)
        k_new = int(math.ceil(0.5 * k))
        xlo, xhi, mask, read = _layer(
            k, k_new, xlo, xhi, alo, ahi, mask, p['conv%d' % i],
            p['bn%d_g' % i], p['bn%d_b' % i], p['pool%d_w' % i])
        k = k_new
        reads.append(read)
    return _final(reads[0], reads[1], reads[2], p['lin1_W'], p['lin1_b'])


# speedup vs baseline: 2.3126x; 1.1325x over previous
import functools
import math

import jax
import jax.numpy as jnp
from jax import lax
from jax.experimental import pallas as pl
from jax.experimental.pallas import tpu as pltpu
from jax.experimental.pallas import tpu_sc as plsc

N = 10000
E = 320000
DF = 128
H = 256
D = 128
NS = 16
NC = 2
CH = 128
BLK = 16
NBLK = 10
CHUNKS = BLK * NBLK
E_PAD = CHUNKS * CH * NS - E
ACC_ROWS = 10112
BNEPS = 1e-5
NEG_HUGE = -3.0e38

@functools.cache
def _sc_mesh():
    return plsc.VectorSubcoreMesh(core_axis_name="c", subcore_axis_name="s",
                                  num_cores=NC, num_subcores=NS)


_ZERO_SL = ACC_ROWS // NS
_OUT_SL = 632
_OUT_SL_LAST = N - 15 * _OUT_SL


def _seg_sum_kernel(xlo_hbm, xhi_hbm, src_hbm, dst_hbm, z_hbm,
                    alo_hbm, ahi_hbm, src_v, dst_v, rows_a, rows_b, acc,
                    sem_a, sem_b):
    c = lax.axis_index("c")
    s = lax.axis_index("s")
    pltpu.sync_copy(z_hbm.at[pl.ds(s * _ZERO_SL, _ZERO_SL)],
                    acc.at[pl.ds(s * _ZERO_SL, _ZERO_SL)])
    plsc.subcore_barrier()

    def run(x_hbm):
        dummy = x_hbm.at[pl.ds(0, CH)]

        def blk_body(b, carry):
            pltpu.sync_copy(src_hbm.at[s, pl.ds(b * BLK, BLK)], src_v)
            pltpu.sync_copy(dst_hbm.at[s, pl.ds(b * BLK, BLK)], dst_v)
            pltpu.async_copy(x_hbm.at[src_v.at[0]], rows_a, sem_a)

            def pair_body(t, inner):
                pltpu.make_async_copy(dummy, rows_a, sem_a).wait()
                pltpu.async_copy(x_hbm.at[src_v.at[2 * t + 1]], rows_b, sem_b)
                pltpu.sync_copy(rows_a, acc.at[dst_v.at[2 * t]], add=True)
                pltpu.make_async_copy(dummy, rows_b, sem_b).wait()

                @pl.when(t < BLK // 2 - 1)
                def _():
                    pltpu.async_copy(x_hbm.at[src_v.at[2 * t + 2]], rows_a,
                                     sem_a)

                pltpu.sync_copy(rows_b, acc.at[dst_v.at[2 * t + 1]], add=True)
                return inner

            return lax.fori_loop(0, BLK // 2, pair_body, carry)

        lax.fori_loop(0, NBLK, blk_body, jnp.int32(0))

    @pl.when(c == 0)
    def _():
        run(xlo_hbm)

    @pl.when(c == 1)
    def _():
        run(xhi_hbm)

    plsc.subcore_barrier()

    def writeout(a_hbm):
        @pl.when(s < 15)
        def _():
            pltpu.sync_copy(acc.at[pl.ds(s * _OUT_SL, _OUT_SL)],
                            a_hbm.at[pl.ds(s * _OUT_SL, _OUT_SL)])

        @pl.when(s == 15)
        def _():
            pltpu.sync_copy(acc.at[pl.ds(15 * _OUT_SL, _OUT_SL_LAST)],
                            a_hbm.at[pl.ds(15 * _OUT_SL, _OUT_SL_LAST)])

    @pl.when(c == 0)
    def _():
        writeout(alo_hbm)

    @pl.when(c == 1)
    def _():
        writeout(ahi_hbm)


def _seg_sum_sc(xlo, xhi, srcp, dstp, zeros):
    return pl.kernel(
        _seg_sum_kernel,
        out_type=(jax.ShapeDtypeStruct((N, D), jnp.float32),
                  jax.ShapeDtypeStruct((N, D), jnp.float32)),
        mesh=_sc_mesh(),
        scratch_types=[
            pltpu.VMEM((BLK, CH), jnp.int32),
            pltpu.VMEM((BLK, CH), jnp.int32),
            pltpu.VMEM((CH, D), jnp.float32),
            pltpu.VMEM((CH, D), jnp.float32),
            pltpu.VMEM_SHARED((ACC_ROWS, D), jnp.float32),
            pltpu.SemaphoreType.DMA,
            pltpu.SemaphoreType.DMA,
        ],
    )(xlo, xhi, srcp, dstp, zeros)


FBLK = 8
FCAP_CH = 176
FCAP = FCAP_CH * CH
_EPB = FBLK * CH


def _edge_filter_kernel(src_hbm, dst_hbm, mask_hbm,
                        fsrc_hbm, fdst_hbm, cnt_hbm,
                        src_v, dst_v, mask_v, osrc_v, odst_v, cnt_v):
    c = lax.axis_index("c")
    s = lax.axis_index("s")

    @pl.when(c == 0)
    def _():
        pltpu.sync_copy(mask_hbm, mask_v)
        idx16 = lax.iota(jnp.int32, 16)

        def blk(b, off):
            pltpu.sync_copy(src_hbm.at[s, pl.ds(b * BLK * CH, BLK * CH)],
                            src_v)
            pltpu.sync_copy(dst_hbm.at[s, pl.ds(b * BLK * CH, BLK * CH)],
                            dst_v)

            def grp(g, off2):
                o = pl.multiple_of(g * 16, 16)
                sv = src_v[pl.ds(o, 16)]
                dv = dst_v[pl.ds(o, 16)]
                ms = plsc.load_gather(mask_v, [sv])
                md = plsc.load_gather(mask_v, [dv])
                ok = (ms > 0.0) & (md > 0.0)
                pos = plsc.cumsum(ok.astype(jnp.int32))
                tgt = off2 + pos - 1
                plsc.store_scatter(osrc_v, [tgt], sv, mask=ok)
                plsc.store_scatter(odst_v, [tgt], dv, mask=ok)
                return off2 + jnp.max(pos)

            return lax.fori_loop(0, BLK * CH // 16, grp, off)

        off = lax.fori_loop(0, NBLK, blk, jnp.int32(0))

        dump16 = N + ((idx16 + s * 16) & 63)

        def fill(t, carry):
            tgt = off + t * 16 + idx16
            srcf = (idx16 * 613 + t * 89 + s * 509) & 8191
            plsc.store_scatter(osrc_v, [tgt], srcf)
            plsc.store_scatter(odst_v, [tgt], dump16)
            return carry

        lax.fori_loop(0, _EPB // 16, fill, jnp.int32(0))
        nblk = (off + _EPB - 1) // _EPB
        cnt_v[...] = jnp.full((16,), nblk, jnp.int32)
        pltpu.sync_copy(osrc_v, fsrc_hbm.at[s])
        pltpu.sync_copy(odst_v, fdst_hbm.at[s])
        pltpu.sync_copy(cnt_v, cnt_hbm.at[s])


def _edge_filter(src2d, dst2d, mask1d):
    return pl.kernel(
        _edge_filter_kernel,
        out_type=(jax.ShapeDtypeStruct((NS, FCAP), jnp.int32),
                  jax.ShapeDtypeStruct((NS, FCAP), jnp.int32),
                  jax.ShapeDtypeStruct((NS, 16), jnp.int32)),
        mesh=_sc_mesh(),
        compiler_params=pltpu.CompilerParams(needs_layout_passes=False),
        scratch_types=[
            pltpu.VMEM((BLK * CH,), jnp.int32),
            pltpu.VMEM((BLK * CH,), jnp.int32),
            pltpu.VMEM((N,), jnp.float32),
            pltpu.VMEM((FCAP,), jnp.int32),
            pltpu.VMEM((FCAP,), jnp.int32),
            pltpu.VMEM((16,), jnp.int32),
        ],
    )(src2d, dst2d, mask1d)


def _seg_sum_f_kernel(xlo_hbm, xhi_hbm, fsrc_hbm, fdst_hbm, cnt_hbm, z_hbm,
                      alo_hbm, ahi_hbm,
                      src_v, dst_v, rows_a, rows_b, cnt_v, acc, sem_a, sem_b):
    c = lax.axis_index("c")
    s = lax.axis_index("s")
    pltpu.sync_copy(z_hbm.at[pl.ds(s * _ZERO_SL, _ZERO_SL)],
                    acc.at[pl.ds(s * _ZERO_SL, _ZERO_SL)])
    pltpu.sync_copy(cnt_hbm.at[s], cnt_v)
    plsc.subcore_barrier()
    nblk = cnt_v[...][0]

    def run(x_hbm):
        dummy = x_hbm.at[pl.ds(0, CH)]

        def blk_body(b, carry):
            pltpu.sync_copy(fsrc_hbm.at[s, pl.ds(b * FBLK, FBLK)], src_v)
            pltpu.sync_copy(fdst_hbm.at[s, pl.ds(b * FBLK, FBLK)], dst_v)
            pltpu.async_copy(x_hbm.at[src_v.at[0]], rows_a, sem_a)

            def pair_body(t, inner):
                pltpu.make_async_copy(dummy, rows_a, sem_a).wait()
                pltpu.async_copy(x_hbm.at[src_v.at[2 * t + 1]], rows_b, sem_b)
                pltpu.sync_copy(rows_a, acc.at[dst_v.at[2 * t]], add=True)
                pltpu.make_async_copy(dummy, rows_b, sem_b).wait()

                @pl.when(t < FBLK // 2 - 1)
                def _():
                    pltpu.async_copy(x_hbm.at[src_v.at[2 * t + 2]], rows_a,
                                     sem_a)

                pltpu.sync_copy(rows_b, acc.at[dst_v.at[2 * t + 1]], add=True)
                return inner

            return lax.fori_loop(0, FBLK // 2, pair_body, carry)

        lax.fori_loop(0, nblk, blk_body, jnp.int32(0))

    @pl.when(c == 0)
    def _():
        run(xlo_hbm)

    @pl.when(c == 1)
    def _():
        run(xhi_hbm)

    plsc.subcore_barrier()

    def writeout(a_hbm):
        @pl.when(s < 15)
        def _():
            pltpu.sync_copy(acc.at[pl.ds(s * _OUT_SL, _OUT_SL)],
                            a_hbm.at[pl.ds(s * _OUT_SL, _OUT_SL)])

        @pl.when(s == 15)
        def _():
            pltpu.sync_copy(acc.at[pl.ds(15 * _OUT_SL, _OUT_SL_LAST)],
                            a_hbm.at[pl.ds(15 * _OUT_SL, _OUT_SL_LAST)])

    @pl.when(c == 0)
    def _():
        writeout(alo_hbm)

    @pl.when(c == 1)
    def _():
        writeout(ahi_hbm)


def _seg_sum_f_sc(xlo, xhi, fsrc, fdst, cnts, zeros):
    return pl.kernel(
        _seg_sum_f_kernel,
        out_type=(jax.ShapeDtypeStruct((N, D), jnp.float32),
                  jax.ShapeDtypeStruct((N, D), jnp.float32)),
        mesh=_sc_mesh(),
        compiler_params=pltpu.CompilerParams(needs_layout_passes=False),
        scratch_types=[
            pltpu.VMEM((FBLK, CH), jnp.int32),
            pltpu.VMEM((FBLK, CH), jnp.int32),
            pltpu.VMEM((CH, D), jnp.float32),
            pltpu.VMEM((CH, D), jnp.float32),
            pltpu.VMEM((16,), jnp.int32),
            pltpu.VMEM_SHARED((ACC_ROWS, D), jnp.float32),
            pltpu.SemaphoreType.DMA,
            pltpu.SemaphoreType.DMA,
        ],
    )(xlo, xhi, fsrc.reshape(NS, FCAP_CH, CH), fdst.reshape(NS, FCAP_CH, CH),
      cnts, zeros)


def _emb_kernel(x_ref, w_ref, b_ref, lo_ref, hi_ref):
    h = jnp.dot(x_ref[...], w_ref[...], preferred_element_type=jnp.float32)
    h = jnp.maximum(h + b_ref[...], 0.0)
    lo_ref[...] = h[:, :D]
    hi_ref[...] = h[:, D:]


def _emb(x, w, b):
    return pl.pallas_call(
        _emb_kernel,
        out_shape=(jax.ShapeDtypeStruct((N, D), jnp.float32),
                   jax.ShapeDtypeStruct((N, D), jnp.float32)),
    )(x, w, b)


def _masked_bn(h, m, k, g, b):
    mu = jnp.sum(h * m, axis=0, keepdims=True) * (1.0 / k)
    d = h - mu
    var = jnp.sum(d * d * m, axis=0, keepdims=True) * (1.0 / k)
    return d * lax.rsqrt(var + BNEPS) * g + b


def _layer_a_kernel(k_prev,
                    xlo_ref, xhi_ref, alo_ref, ahi_ref, m_ref,
                    w1_ref, b1_ref, g1_ref, bb1_ref,
                    hlo_ref, hhi_ref):
    m = m_ref[...]
    h = (jnp.dot(xlo_ref[...] + alo_ref[...], w1_ref[:D, :],
                 preferred_element_type=jnp.float32)
         + jnp.dot(xhi_ref[...] + ahi_ref[...], w1_ref[D:, :],
                   preferred_element_type=jnp.float32)
         + b1_ref[...])
    h = jnp.maximum(_masked_bn(h, m, k_prev, g1_ref[...], bb1_ref[...]), 0.0)
    hlo_ref[...] = h[:, :D]
    hhi_ref[...] = h[:, D:]


def _layer_b_kernel(k_prev,
                    hlo_ref, hhi_ref, m_ref,
                    w2_ref, b2_ref, g2_ref, bb2_ref, pw_ref,
                    ylo_ref, yhi_ref, sc_ref):
    m = m_ref[...]
    h = (jnp.dot(hlo_ref[...], w2_ref[:D, :],
                 preferred_element_type=jnp.float32)
         + jnp.dot(hhi_ref[...], w2_ref[D:, :],
                   preferred_element_type=jnp.float32)
         + b2_ref[...])
    y = _masked_bn(h, m, k_prev, g2_ref[...], bb2_ref[...])
    y = jnp.where(y > 0, y, 0.1 * y)

    pw = pw_ref[...]
    nrm = jnp.sqrt(jnp.sum(pw * pw)) + 1e-16
    score = jnp.tanh(jnp.dot(y, pw, preferred_element_type=jnp.float32) / nrm)
    ylo_ref[...] = y[:, :D]
    yhi_ref[...] = y[:, D:]
    sc_ref[...] = score


def _layer_c_kernel(k_new,
                    ylo_ref, yhi_ref, sc_ref, scr_ref, mr_ref, mc_ref,
                    xnlo_ref, xnhi_ref, mn_ref, read_ref):
    score_r = scr_ref[...]
    m_r = mr_ref[...]
    sm = jnp.where(m_r > 0, score_r, -2.0)

    u = lax.bitcast_convert_type(sm, jnp.uint32)
    ukey = jnp.where((u >> 31) != 0, ~u, u | jnp.uint32(0x80000000))

    def t_body(i, prefix):
        cand = prefix | (jnp.uint32(1) << (31 - i).astype(jnp.uint32))
        cnt = jnp.sum((ukey >= cand).astype(jnp.int32))
        return jnp.where(cnt >= k_new, cand, prefix)

    tkey = lax.fori_loop(0, 32, t_body, jnp.uint32(0))
    c_gt = jnp.sum((ukey > tkey).astype(jnp.int32))
    mrem = k_new - c_gt

    def r_body(i, prefix):
        bit = jnp.int32(1) << (13 - i).astype(jnp.int32)
        cap = prefix | (bit - 1)
        idx = lax.broadcasted_iota(jnp.int32, (1, N), 1)
        cnt = jnp.sum(((ukey == tkey) & (idx <= cap)).astype(jnp.int32))
        return jnp.where(cnt >= mrem, prefix, prefix | bit)

    ridx = lax.fori_loop(0, 14, r_body, jnp.int32(0))

    score = sc_ref[...]
    mc = lax.bitcast_convert_type(jnp.where(mc_ref[...] > 0, score, -2.0),
                                  jnp.uint32)
    ukc = jnp.where((mc >> 31) != 0, ~mc, mc | jnp.uint32(0x80000000))
    idc = lax.broadcasted_iota(jnp.int32, (N, 1), 0)
    sel = (ukc > tkey) | ((ukc == tkey) & (idc <= ridx) & (mrem > 0))
    mn = sel.astype(jnp.float32)
    mn_ref[...] = mn
    sc = score * mn
    xnlo = ylo_ref[...] * sc
    xnhi = yhi_ref[...] * sc
    xnlo_ref[...] = xnlo
    xnhi_ref[...] = xnhi
    mxlo = jnp.max(jnp.where(sel, xnlo, NEG_HUGE), axis=0, keepdims=True)
    mxhi = jnp.max(jnp.where(sel, xnhi, NEG_HUGE), axis=0, keepdims=True)
    mnlo = jnp.sum(xnlo, axis=0, keepdims=True) * (1.0 / k_new)
    mnhi = jnp.sum(xnhi, axis=0, keepdims=True) * (1.0 / k_new)
    read_ref[...] = jnp.concatenate([mxlo, mxhi, mnlo, mnhi], axis=1)


def _layer(k_prev, k_new, xlo, xhi, alo, ahi, mask, cp, g, b, pw):
    hlo, hhi = pl.pallas_call(
        functools.partial(_layer_a_kernel, k_prev),
        out_shape=(jax.ShapeDtypeStruct((N, D), jnp.float32),
                   jax.ShapeDtypeStruct((N, D), jnp.float32)),
    )(xlo, xhi, alo, ahi, mask,
      cp['lin1_W'], cp['lin1_b'].reshape(1, H), cp['bn_g'].reshape(1, H),
      cp['bn_b'].reshape(1, H))
    ylo, yhi, score = pl.pallas_call(
        functools.partial(_layer_b_kernel, k_prev),
        out_shape=(jax.ShapeDtypeStruct((N, D), jnp.float32),
                   jax.ShapeDtypeStruct((N, D), jnp.float32),
                   jax.ShapeDtypeStruct((N, 1), jnp.float32)),
    )(hlo, hhi, mask,
      cp['lin2_W'], cp['lin2_b'].reshape(1, H),
      g.reshape(1, H), b.reshape(1, H), pw.reshape(H, 1))
    score_row = score.reshape(1, N)
    mask_row = mask.reshape(1, N)
    return pl.pallas_call(
        functools.partial(_layer_c_kernel, k_new),
        out_shape=(jax.ShapeDtypeStruct((N, D), jnp.float32),
                   jax.ShapeDtypeStruct((N, D), jnp.float32),
                   jax.ShapeDtypeStruct((N, 1), jnp.float32),
                   jax.ShapeDtypeStruct((1, 2 * H), jnp.float32)),
    )(ylo, yhi, score, score_row, mask_row, mask)


def _final_kernel(r1_ref, r2_ref, r3_ref, w_ref, b_ref, out_ref):
    def lk(v):
        return jnp.where(v > 0, v, 0.1 * v)

    agg = lk(r1_ref[...]) + lk(r2_ref[...]) + lk(r3_ref[...])
    out_ref[...] = (jnp.dot(agg, w_ref[...], preferred_element_type=jnp.float32)
                    + b_ref[...])


def _final(r1, r2, r3, w, b):
    return pl.pallas_call(
        _final_kernel,
        out_shape=jax.ShapeDtypeStruct((1, H), jnp.float32),
    )(r1, r2, r3, w, b.reshape(1, H))


def kernel(x, edge_index, batch, params):
    p = params
    src = edge_index[0].astype(jnp.int32)
    dst = edge_index[1].astype(jnp.int32)
    src2d = jnp.concatenate(
        [src, (jnp.arange(E_PAD, dtype=jnp.int32) * 613) & 8191]
    ).reshape(NS, CHUNKS * CH)
    dst2d = jnp.concatenate(
        [dst, N + (jnp.arange(E_PAD, dtype=jnp.int32) % 64)]
    ).reshape(NS, CHUNKS * CH)
    srcp = src2d.reshape(NS, CHUNKS, CH)
    dstp = dst2d.reshape(NS, CHUNKS, CH)
    zeros = jnp.zeros((ACC_ROWS, D), jnp.float32)

    xlo, xhi = _emb(x, p['emb_W'], p['emb_b'].reshape(1, H))
    mask = jnp.ones((N, 1), jnp.float32)
    k = N
    reads = []
    for i in (1, 2, 3):
        if i == 1:
            alo, ahi = _seg_sum_sc(xlo, xhi, srcp, dstp, zeros)
        else:
            fsrc, fdst, cnts = _edge_filter(src2d, dst2d, mask.reshape(N))
            alo, ahi = _seg_sum_f_sc(xlo, xhi, fsrc, fdst, cnts, zeros)
        k_new = int(math.ceil(0.5 * k))
        xlo, xhi, mask, read = _layer(
            k, k_new, xlo, xhi, alo, ahi, mask, p['conv%d' % i],
            p['bn%d_g' % i], p['bn%d_b' % i], p['pool%d_w' % i])
        k = k_new
        reads.append(read)
    return _final(reads[0], reads[1], reads[2], p['lin1_W'], p['lin1_b'])


# speedup vs baseline: 2.4195x; 1.0462x over previous
import functools
import math

import jax
import jax.numpy as jnp
from jax import lax
from jax.experimental import pallas as pl
from jax.experimental.pallas import tpu as pltpu
from jax.experimental.pallas import tpu_sc as plsc

N = 10000
E = 320000
DF = 128
H = 256
D = 128
NS = 16
NC = 2
CH = 128
BLK = 16
NBLK = 10
CHUNKS = BLK * NBLK
E_PAD = CHUNKS * CH * NS - E
ACC_ROWS = 10112
BNEPS = 1e-5
NEG_HUGE = -3.0e38

@functools.cache
def _sc_mesh():
    return plsc.VectorSubcoreMesh(core_axis_name="c", subcore_axis_name="s",
                                  num_cores=NC, num_subcores=NS)


_ZERO_SL = ACC_ROWS // NS
_OUT_SL = 632
_OUT_SL_LAST = N - 15 * _OUT_SL


def _seg_sum_kernel(xlo_hbm, xhi_hbm, src_hbm, dst_hbm, z_hbm,
                    alo_hbm, ahi_hbm, src_v, dst_v, rows_a, rows_b, acc,
                    sem_a, sem_b):
    c = lax.axis_index("c")
    s = lax.axis_index("s")
    pltpu.sync_copy(z_hbm.at[pl.ds(s * _ZERO_SL, _ZERO_SL)],
                    acc.at[pl.ds(s * _ZERO_SL, _ZERO_SL)])
    plsc.subcore_barrier()

    def run(x_hbm):
        dummy = x_hbm.at[pl.ds(0, CH)]

        def blk_body(b, carry):
            pltpu.sync_copy(src_hbm.at[s, pl.ds(b * BLK, BLK)], src_v)
            pltpu.sync_copy(dst_hbm.at[s, pl.ds(b * BLK, BLK)], dst_v)
            pltpu.async_copy(x_hbm.at[src_v.at[0]], rows_a, sem_a)

            def pair_body(t, inner):
                pltpu.make_async_copy(dummy, rows_a, sem_a).wait()
                pltpu.async_copy(x_hbm.at[src_v.at[2 * t + 1]], rows_b, sem_b)
                pltpu.sync_copy(rows_a, acc.at[dst_v.at[2 * t]], add=True)
                pltpu.make_async_copy(dummy, rows_b, sem_b).wait()

                @pl.when(t < BLK // 2 - 1)
                def _():
                    pltpu.async_copy(x_hbm.at[src_v.at[2 * t + 2]], rows_a,
                                     sem_a)

                pltpu.sync_copy(rows_b, acc.at[dst_v.at[2 * t + 1]], add=True)
                return inner

            return lax.fori_loop(0, BLK // 2, pair_body, carry)

        lax.fori_loop(0, NBLK, blk_body, jnp.int32(0))

    @pl.when(c == 0)
    def _():
        run(xlo_hbm)

    @pl.when(c == 1)
    def _():
        run(xhi_hbm)

    plsc.subcore_barrier()

    def writeout(a_hbm):
        @pl.when(s < 15)
        def _():
            pltpu.sync_copy(acc.at[pl.ds(s * _OUT_SL, _OUT_SL)],
                            a_hbm.at[pl.ds(s * _OUT_SL, _OUT_SL)])

        @pl.when(s == 15)
        def _():
            pltpu.sync_copy(acc.at[pl.ds(15 * _OUT_SL, _OUT_SL_LAST)],
                            a_hbm.at[pl.ds(15 * _OUT_SL, _OUT_SL_LAST)])

    @pl.when(c == 0)
    def _():
        writeout(alo_hbm)

    @pl.when(c == 1)
    def _():
        writeout(ahi_hbm)


def _seg_sum_sc(xlo, xhi, srcp, dstp, zeros):
    return pl.kernel(
        _seg_sum_kernel,
        out_type=(jax.ShapeDtypeStruct((N, D), jnp.float32),
                  jax.ShapeDtypeStruct((N, D), jnp.float32)),
        mesh=_sc_mesh(),
        scratch_types=[
            pltpu.VMEM((BLK, CH), jnp.int32),
            pltpu.VMEM((BLK, CH), jnp.int32),
            pltpu.VMEM((CH, D), jnp.float32),
            pltpu.VMEM((CH, D), jnp.float32),
            pltpu.VMEM_SHARED((ACC_ROWS, D), jnp.float32),
            pltpu.SemaphoreType.DMA,
            pltpu.SemaphoreType.DMA,
        ],
    )(xlo, xhi, srcp, dstp, zeros)


FBLK = 8
FCAP_CH = 176
FCAP = FCAP_CH * CH
_EPB = FBLK * CH


def _edge_filter_kernel(src_hbm, dst_hbm, mask_hbm,
                        fsrc_hbm, fdst_hbm, cnt_hbm,
                        src_v, dst_v, mask_v, osrc_v, odst_v, cnt_v):
    c = lax.axis_index("c")
    s = lax.axis_index("s")

    @pl.when(c == 0)
    def _():
        pltpu.sync_copy(mask_hbm, mask_v)
        idx16 = lax.iota(jnp.int32, 16)

        def blk(b, off):
            pltpu.sync_copy(src_hbm.at[s, pl.ds(b * BLK * CH, BLK * CH)],
                            src_v)
            pltpu.sync_copy(dst_hbm.at[s, pl.ds(b * BLK * CH, BLK * CH)],
                            dst_v)

            def grp(g, off2):
                o = pl.multiple_of(g * 16, 16)
                sv = src_v[pl.ds(o, 16)]
                dv = dst_v[pl.ds(o, 16)]
                ms = plsc.load_gather(mask_v, [sv])
                md = plsc.load_gather(mask_v, [dv])
                ok = (ms > 0.0) & (md > 0.0)
                pos = plsc.cumsum(ok.astype(jnp.int32))
                tgt = off2 + pos - 1
                plsc.store_scatter(osrc_v, [tgt], sv, mask=ok)
                plsc.store_scatter(odst_v, [tgt], dv, mask=ok)
                return off2 + jnp.max(pos)

            return lax.fori_loop(0, BLK * CH // 16, grp, off)

        off = lax.fori_loop(0, NBLK, blk, jnp.int32(0))

        dump16 = N + ((idx16 + s * 16) & 63)

        def fill(t, carry):
            tgt = off + t * 16 + idx16
            srcf = (idx16 * 613 + t * 89 + s * 509) & 8191
            plsc.store_scatter(osrc_v, [tgt], srcf)
            plsc.store_scatter(odst_v, [tgt], dump16)
            return carry

        lax.fori_loop(0, _EPB // 16, fill, jnp.int32(0))
        nblk = (off + _EPB - 1) // _EPB
        cnt_v[...] = jnp.full((16,), nblk, jnp.int32)
        pltpu.sync_copy(osrc_v, fsrc_hbm.at[s])
        pltpu.sync_copy(odst_v, fdst_hbm.at[s])
        pltpu.sync_copy(cnt_v, cnt_hbm.at[s])


def _edge_filter(src2d, dst2d, mask1d):
    return pl.kernel(
        _edge_filter_kernel,
        out_type=(jax.ShapeDtypeStruct((NS, FCAP), jnp.int32),
                  jax.ShapeDtypeStruct((NS, FCAP), jnp.int32),
                  jax.ShapeDtypeStruct((NS, 16), jnp.int32)),
        mesh=_sc_mesh(),
        compiler_params=pltpu.CompilerParams(needs_layout_passes=False),
        scratch_types=[
            pltpu.VMEM((BLK * CH,), jnp.int32),
            pltpu.VMEM((BLK * CH,), jnp.int32),
            pltpu.VMEM((N,), jnp.float32),
            pltpu.VMEM((FCAP,), jnp.int32),
            pltpu.VMEM((FCAP,), jnp.int32),
            pltpu.VMEM((16,), jnp.int32),
        ],
    )(src2d, dst2d, mask1d)


def _seg_sum_f_kernel(xlo_hbm, xhi_hbm, fsrc_hbm, fdst_hbm, cnt_hbm, z_hbm,
                      alo_hbm, ahi_hbm,
                      src_v, dst_v, rows_a, rows_b, cnt_v, acc, sem_a, sem_b):
    c = lax.axis_index("c")
    s = lax.axis_index("s")
    pltpu.sync_copy(z_hbm.at[pl.ds(s * _ZERO_SL, _ZERO_SL)],
                    acc.at[pl.ds(s * _ZERO_SL, _ZERO_SL)])
    pltpu.sync_copy(cnt_hbm.at[s], cnt_v)
    plsc.subcore_barrier()
    nblk = cnt_v[...][0]

    def run(x_hbm):
        dummy = x_hbm.at[pl.ds(0, CH)]

        def blk_body(b, carry):
            pltpu.sync_copy(fsrc_hbm.at[s, pl.ds(b * FBLK, FBLK)], src_v)
            pltpu.sync_copy(fdst_hbm.at[s, pl.ds(b * FBLK, FBLK)], dst_v)
            pltpu.async_copy(x_hbm.at[src_v.at[0]], rows_a, sem_a)

            def pair_body(t, inner):
                pltpu.make_async_copy(dummy, rows_a, sem_a).wait()
                pltpu.async_copy(x_hbm.at[src_v.at[2 * t + 1]], rows_b, sem_b)
                pltpu.sync_copy(rows_a, acc.at[dst_v.at[2 * t]], add=True)
                pltpu.make_async_copy(dummy, rows_b, sem_b).wait()

                @pl.when(t < FBLK // 2 - 1)
                def _():
                    pltpu.async_copy(x_hbm.at[src_v.at[2 * t + 2]], rows_a,
                                     sem_a)

                pltpu.sync_copy(rows_b, acc.at[dst_v.at[2 * t + 1]], add=True)
                return inner

            return lax.fori_loop(0, FBLK // 2, pair_body, carry)

        lax.fori_loop(0, nblk, blk_body, jnp.int32(0))

    @pl.when(c == 0)
    def _():
        run(xlo_hbm)

    @pl.when(c == 1)
    def _():
        run(xhi_hbm)

    plsc.subcore_barrier()

    def writeout(a_hbm):
        @pl.when(s < 15)
        def _():
            pltpu.sync_copy(acc.at[pl.ds(s * _OUT_SL, _OUT_SL)],
                            a_hbm.at[pl.ds(s * _OUT_SL, _OUT_SL)])

        @pl.when(s == 15)
        def _():
            pltpu.sync_copy(acc.at[pl.ds(15 * _OUT_SL, _OUT_SL_LAST)],
                            a_hbm.at[pl.ds(15 * _OUT_SL, _OUT_SL_LAST)])

    @pl.when(c == 0)
    def _():
        writeout(alo_hbm)

    @pl.when(c == 1)
    def _():
        writeout(ahi_hbm)


def _seg_sum_f_sc(xlo, xhi, fsrc, fdst, cnts, zeros):
    return pl.kernel(
        _seg_sum_f_kernel,
        out_type=(jax.ShapeDtypeStruct((N, D), jnp.float32),
                  jax.ShapeDtypeStruct((N, D), jnp.float32)),
        mesh=_sc_mesh(),
        compiler_params=pltpu.CompilerParams(needs_layout_passes=False),
        scratch_types=[
            pltpu.VMEM((FBLK, CH), jnp.int32),
            pltpu.VMEM((FBLK, CH), jnp.int32),
            pltpu.VMEM((CH, D), jnp.float32),
            pltpu.VMEM((CH, D), jnp.float32),
            pltpu.VMEM((16,), jnp.int32),
            pltpu.VMEM_SHARED((ACC_ROWS, D), jnp.float32),
            pltpu.SemaphoreType.DMA,
            pltpu.SemaphoreType.DMA,
        ],
    )(xlo, xhi, fsrc.reshape(NS, FCAP_CH, CH), fdst.reshape(NS, FCAP_CH, CH),
      cnts, zeros)


def _emb_kernel(x_ref, w_ref, b_ref, lo_ref, hi_ref):
    h = jnp.dot(x_ref[...], w_ref[...], preferred_element_type=jnp.float32)
    h = jnp.maximum(h + b_ref[...], 0.0)
    lo_ref[...] = h[:, :D]
    hi_ref[...] = h[:, D:]


def _emb(x, w, b):
    return pl.pallas_call(
        _emb_kernel,
        out_shape=(jax.ShapeDtypeStruct((N, D), jnp.float32),
                   jax.ShapeDtypeStruct((N, D), jnp.float32)),
    )(x, w, b)


def _masked_bn(h, m, k, g, b):
    mu = jnp.sum(h * m, axis=0, keepdims=True) * (1.0 / k)
    d = h - mu
    var = jnp.sum(d * d * m, axis=0, keepdims=True) * (1.0 / k)
    return d * lax.rsqrt(var + BNEPS) * g + b


def _layer_ab_kernel(k_prev,
                     xlo_ref, xhi_ref, alo_ref, ahi_ref, m_ref,
                     w1_ref, b1_ref, g1_ref, bb1_ref,
                     w2_ref, b2_ref, g2_ref, bb2_ref, pw_ref,
                     ylo_ref, yhi_ref, sc_ref):
    m = m_ref[...]
    h = (jnp.dot(xlo_ref[...] + alo_ref[...], w1_ref[:D, :],
                 preferred_element_type=jnp.float32)
         + jnp.dot(xhi_ref[...] + ahi_ref[...], w1_ref[D:, :],
                   preferred_element_type=jnp.float32)
         + b1_ref[...])
    h = jnp.maximum(_masked_bn(h, m, k_prev, g1_ref[...], bb1_ref[...]), 0.0)
    h = (jnp.dot(h, w2_ref[...], preferred_element_type=jnp.float32)
         + b2_ref[...])
    y = _masked_bn(h, m, k_prev, g2_ref[...], bb2_ref[...])
    y = jnp.where(y > 0, y, 0.1 * y)

    pw = pw_ref[...]
    nrm = jnp.sqrt(jnp.sum(pw * pw)) + 1e-16
    score = jnp.tanh(jnp.dot(y, pw, preferred_element_type=jnp.float32) / nrm)
    ylo_ref[...] = y[:, :D]
    yhi_ref[...] = y[:, D:]
    sc_ref[...] = score


def _layer_c_kernel(k_new,
                    ylo_ref, yhi_ref, sc_ref, scr_ref, mr_ref, mc_ref,
                    xnlo_ref, xnhi_ref, mn_ref, read_ref):
    score_r = scr_ref[...]
    m_r = mr_ref[...]
    sm = jnp.where(m_r > 0, score_r, -2.0)

    u = lax.bitcast_convert_type(sm, jnp.uint32)
    ukey = jnp.where((u >> 31) != 0, ~u, u | jnp.uint32(0x80000000))

    def t_body(i, prefix):
        cand = prefix | (jnp.uint32(1) << (31 - i).astype(jnp.uint32))
        cnt = jnp.sum((ukey >= cand).astype(jnp.int32))
        return jnp.where(cnt >= k_new, cand, prefix)

    tkey = lax.fori_loop(0, 32, t_body, jnp.uint32(0))
    c_gt = jnp.sum((ukey > tkey).astype(jnp.int32))
    mrem = k_new - c_gt

    def r_body(i, prefix):
        bit = jnp.int32(1) << (13 - i).astype(jnp.int32)
        cap = prefix | (bit - 1)
        idx = lax.broadcasted_iota(jnp.int32, (1, N), 1)
        cnt = jnp.sum(((ukey == tkey) & (idx <= cap)).astype(jnp.int32))
        return jnp.where(cnt >= mrem, prefix, prefix | bit)

    ridx = lax.fori_loop(0, 14, r_body, jnp.int32(0))

    score = sc_ref[...]
    mc = lax.bitcast_convert_type(jnp.where(mc_ref[...] > 0, score, -2.0),
                                  jnp.uint32)
    ukc = jnp.where((mc >> 31) != 0, ~mc, mc | jnp.uint32(0x80000000))
    idc = lax.broadcasted_iota(jnp.int32, (N, 1), 0)
    sel = (ukc > tkey) | ((ukc == tkey) & (idc <= ridx) & (mrem > 0))
    mn = sel.astype(jnp.float32)
    mn_ref[...] = mn
    sc = score * mn
    xnlo = ylo_ref[...] * sc
    xnhi = yhi_ref[...] * sc
    xnlo_ref[...] = xnlo
    xnhi_ref[...] = xnhi
    mxlo = jnp.max(jnp.where(sel, xnlo, NEG_HUGE), axis=0, keepdims=True)
    mxhi = jnp.max(jnp.where(sel, xnhi, NEG_HUGE), axis=0, keepdims=True)
    mnlo = jnp.sum(xnlo, axis=0, keepdims=True) * (1.0 / k_new)
    mnhi = jnp.sum(xnhi, axis=0, keepdims=True) * (1.0 / k_new)
    read_ref[...] = jnp.concatenate([mxlo, mxhi, mnlo, mnhi], axis=1)


def _layer(k_prev, k_new, xlo, xhi, alo, ahi, mask, cp, g, b, pw):
    ylo, yhi, score = pl.pallas_call(
        functools.partial(_layer_ab_kernel, k_prev),
        out_shape=(jax.ShapeDtypeStruct((N, D), jnp.float32),
                   jax.ShapeDtypeStruct((N, D), jnp.float32),
                   jax.ShapeDtypeStruct((N, 1), jnp.float32)),
        compiler_params=pltpu.CompilerParams(
            vmem_limit_bytes=64 * 1024 * 1024),
    )(xlo, xhi, alo, ahi, mask,
      cp['lin1_W'], cp['lin1_b'].reshape(1, H), cp['bn_g'].reshape(1, H),
      cp['bn_b'].reshape(1, H),
      cp['lin2_W'], cp['lin2_b'].reshape(1, H),
      g.reshape(1, H), b.reshape(1, H), pw.reshape(H, 1))
    score_row = score.reshape(1, N)
    mask_row = mask.reshape(1, N)
    return pl.pallas_call(
        functools.partial(_layer_c_kernel, k_new),
        out_shape=(jax.ShapeDtypeStruct((N, D), jnp.float32),
                   jax.ShapeDtypeStruct((N, D), jnp.float32),
                   jax.ShapeDtypeStruct((N, 1), jnp.float32),
                   jax.ShapeDtypeStruct((1, 2 * H), jnp.float32)),
    )(ylo, yhi, score, score_row, mask_row, mask)


def _final_kernel(r1_ref, r2_ref, r3_ref, w_ref, b_ref, out_ref):
    def lk(v):
        return jnp.where(v > 0, v, 0.1 * v)

    agg = lk(r1_ref[...]) + lk(r2_ref[...]) + lk(r3_ref[...])
    out_ref[...] = (jnp.dot(agg, w_ref[...], preferred_element_type=jnp.float32)
                    + b_ref[...])


def _final(r1, r2, r3, w, b):
    return pl.pallas_call(
        _final_kernel,
        out_shape=jax.ShapeDtypeStruct((1, H), jnp.float32),
    )(r1, r2, r3, w, b.reshape(1, H))


def kernel(x, edge_index, batch, params):
    p = params
    src = edge_index[0].astype(jnp.int32)
    dst = edge_index[1].astype(jnp.int32)
    src2d = jnp.concatenate(
        [src, (jnp.arange(E_PAD, dtype=jnp.int32) * 613) & 8191]
    ).reshape(NS, CHUNKS * CH)
    dst2d = jnp.concatenate(
        [dst, N + (jnp.arange(E_PAD, dtype=jnp.int32) % 64)]
    ).reshape(NS, CHUNKS * CH)
    srcp = src2d.reshape(NS, CHUNKS, CH)
    dstp = dst2d.reshape(NS, CHUNKS, CH)
    zeros = jnp.zeros((ACC_ROWS, D), jnp.float32)

    xlo, xhi = _emb(x, p['emb_W'], p['emb_b'].reshape(1, H))
    mask = jnp.ones((N, 1), jnp.float32)
    k = N
    reads = []
    for i in (1, 2, 3):
        if i == 1:
            alo, ahi = _seg_sum_sc(xlo, xhi, srcp, dstp, zeros)
        else:
            fsrc, fdst, cnts = _edge_filter(src2d, dst2d, mask.reshape(N))
            alo, ahi = _seg_sum_f_sc(xlo, xhi, fsrc, fdst, cnts, zeros)
        k_new = int(math.ceil(0.5 * k))
        xlo, xhi, mask, read = _layer(
            k, k_new, xlo, xhi, alo, ahi, mask, p['conv%d' % i],
            p['bn%d_g' % i], p['bn%d_b' % i], p['pool%d_w' % i])
        k = k_new
        reads.append(read)
    return _final(reads[0], reads[1], reads[2], p['lin1_W'], p['lin1_b'])


# speedup vs baseline: 2.4897x; 1.0290x over previous
import functools
import math

import jax
import jax.numpy as jnp
from jax import lax
from jax.experimental import pallas as pl
from jax.experimental.pallas import tpu as pltpu
from jax.experimental.pallas import tpu_sc as plsc

N = 10000
E = 320000
DF = 128
H = 256
D = 128
NS = 16
NC = 2
CH = 128
BLK = 32
NBLK = 5
CHUNKS = BLK * NBLK
E_PAD = CHUNKS * CH * NS - E
ACC_ROWS = 10112
BNEPS = 1e-5
NEG_HUGE = -3.0e38

@functools.cache
def _sc_mesh():
    return plsc.VectorSubcoreMesh(core_axis_name="c", subcore_axis_name="s",
                                  num_cores=NC, num_subcores=NS)


_ZERO_SL = ACC_ROWS // NS
_OUT_SL = 632
_OUT_SL_LAST = N - 15 * _OUT_SL


def _seg_sum_kernel(xlo_hbm, xhi_hbm, src_hbm, dst_hbm, z_hbm,
                    alo_hbm, ahi_hbm, src_v, dst_v, rows_a, rows_b, acc,
                    sem_a, sem_b):
    c = lax.axis_index("c")
    s = lax.axis_index("s")
    pltpu.sync_copy(z_hbm.at[pl.ds(s * _ZERO_SL, _ZERO_SL)],
                    acc.at[pl.ds(s * _ZERO_SL, _ZERO_SL)])
    plsc.subcore_barrier()

    def run(x_hbm):
        dummy = x_hbm.at[pl.ds(0, CH)]

        def blk_body(b, carry):
            pltpu.sync_copy(src_hbm.at[s, pl.ds(b * BLK, BLK)], src_v)
            pltpu.sync_copy(dst_hbm.at[s, pl.ds(b * BLK, BLK)], dst_v)
            pltpu.async_copy(x_hbm.at[src_v.at[0]], rows_a, sem_a)

            def pair_body(t, inner):
                pltpu.make_async_copy(dummy, rows_a, sem_a).wait()
                pltpu.async_copy(x_hbm.at[src_v.at[2 * t + 1]], rows_b, sem_b)
                pltpu.sync_copy(rows_a, acc.at[dst_v.at[2 * t]], add=True)
                pltpu.make_async_copy(dummy, rows_b, sem_b).wait()

                @pl.when(t < BLK // 2 - 1)
                def _():
                    pltpu.async_copy(x_hbm.at[src_v.at[2 * t + 2]], rows_a,
                                     sem_a)

                pltpu.sync_copy(rows_b, acc.at[dst_v.at[2 * t + 1]], add=True)
                return inner

            return lax.fori_loop(0, BLK // 2, pair_body, carry)

        lax.fori_loop(0, NBLK, blk_body, jnp.int32(0))

    @pl.when(c == 0)
    def _():
        run(xlo_hbm)

    @pl.when(c == 1)
    def _():
        run(xhi_hbm)

    plsc.subcore_barrier()

    def writeout(a_hbm):
        @pl.when(s < 15)
        def _():
            pltpu.sync_copy(acc.at[pl.ds(s * _OUT_SL, _OUT_SL)],
                            a_hbm.at[pl.ds(s * _OUT_SL, _OUT_SL)])

        @pl.when(s == 15)
        def _():
            pltpu.sync_copy(acc.at[pl.ds(15 * _OUT_SL, _OUT_SL_LAST)],
                            a_hbm.at[pl.ds(15 * _OUT_SL, _OUT_SL_LAST)])

    @pl.when(c == 0)
    def _():
        writeout(alo_hbm)

    @pl.when(c == 1)
    def _():
        writeout(ahi_hbm)


def _seg_sum_sc(xlo, xhi, srcp, dstp, zeros):
    return pl.kernel(
        _seg_sum_kernel,
        out_type=(jax.ShapeDtypeStruct((N, D), jnp.float32),
                  jax.ShapeDtypeStruct((N, D), jnp.float32)),
        mesh=_sc_mesh(),
        scratch_types=[
            pltpu.VMEM((BLK, CH), jnp.int32),
            pltpu.VMEM((BLK, CH), jnp.int32),
            pltpu.VMEM((CH, D), jnp.float32),
            pltpu.VMEM((CH, D), jnp.float32),
            pltpu.VMEM_SHARED((ACC_ROWS, D), jnp.float32),
            pltpu.SemaphoreType.DMA,
            pltpu.SemaphoreType.DMA,
        ],
    )(xlo, xhi, srcp, dstp, zeros)


FBLK = 8
FCAP_CH = 176
FCAP = FCAP_CH * CH
_EPB = FBLK * CH


def _edge_filter_kernel(src_hbm, dst_hbm, mask_hbm,
                        fsrc_hbm, fdst_hbm, cnt_hbm,
                        src_v, dst_v, mask_v, osrc_v, odst_v, cnt_v):
    c = lax.axis_index("c")
    s = lax.axis_index("s")

    @pl.when(c == 0)
    def _():
        pltpu.sync_copy(mask_hbm, mask_v)
        idx16 = lax.iota(jnp.int32, 16)

        def blk(b, off):
            pltpu.sync_copy(src_hbm.at[s, pl.ds(b * BLK * CH, BLK * CH)],
                            src_v)
            pltpu.sync_copy(dst_hbm.at[s, pl.ds(b * BLK * CH, BLK * CH)],
                            dst_v)

            def grp(g, off2):
                o = pl.multiple_of(g * 16, 16)
                sv = src_v[pl.ds(o, 16)]
                dv = dst_v[pl.ds(o, 16)]
                ms = plsc.load_gather(mask_v, [sv])
                md = plsc.load_gather(mask_v, [dv])
                ok = (ms > 0.0) & (md > 0.0)
                pos = plsc.cumsum(ok.astype(jnp.int32))
                tgt = off2 + pos - 1
                plsc.store_scatter(osrc_v, [tgt], sv, mask=ok)
                plsc.store_scatter(odst_v, [tgt], dv, mask=ok)
                return off2 + jnp.max(pos)

            return lax.fori_loop(0, BLK * CH // 16, grp, off)

        off = lax.fori_loop(0, NBLK, blk, jnp.int32(0))

        dump16 = N + ((idx16 + s * 16) & 63)

        def fill(t, carry):
            tgt = off + t * 16 + idx16
            srcf = (idx16 * 613 + t * 89 + s * 509) & 8191
            plsc.store_scatter(osrc_v, [tgt], srcf)
            plsc.store_scatter(odst_v, [tgt], dump16)
            return carry

        lax.fori_loop(0, _EPB // 16, fill, jnp.int32(0))
        nblk = (off + _EPB - 1) // _EPB
        cnt_v[...] = jnp.full((16,), nblk, jnp.int32)
        pltpu.sync_copy(osrc_v, fsrc_hbm.at[s])
        pltpu.sync_copy(odst_v, fdst_hbm.at[s])
        pltpu.sync_copy(cnt_v, cnt_hbm.at[s])


def _edge_filter(src2d, dst2d, mask1d):
    return pl.kernel(
        _edge_filter_kernel,
        out_type=(jax.ShapeDtypeStruct((NS, FCAP), jnp.int32),
                  jax.ShapeDtypeStruct((NS, FCAP), jnp.int32),
                  jax.ShapeDtypeStruct((NS, 16), jnp.int32)),
        mesh=_sc_mesh(),
        compiler_params=pltpu.CompilerParams(needs_layout_passes=False),
        scratch_types=[
            pltpu.VMEM((BLK * CH,), jnp.int32),
            pltpu.VMEM((BLK * CH,), jnp.int32),
            pltpu.VMEM((N,), jnp.float32),
            pltpu.VMEM((FCAP,), jnp.int32),
            pltpu.VMEM((FCAP,), jnp.int32),
            pltpu.VMEM((16,), jnp.int32),
        ],
    )(src2d, dst2d, mask1d)


def _seg_sum_f_kernel(xlo_hbm, xhi_hbm, fsrc_hbm, fdst_hbm, cnt_hbm, z_hbm,
                      alo_hbm, ahi_hbm,
                      src_v, dst_v, rows_a, rows_b, cnt_v, acc, sem_a, sem_b):
    c = lax.axis_index("c")
    s = lax.axis_index("s")
    pltpu.sync_copy(z_hbm.at[pl.ds(s * _ZERO_SL, _ZERO_SL)],
                    acc.at[pl.ds(s * _ZERO_SL, _ZERO_SL)])
    pltpu.sync_copy(cnt_hbm.at[s], cnt_v)
    plsc.subcore_barrier()
    nblk = cnt_v[...][0]

    def run(x_hbm):
        dummy = x_hbm.at[pl.ds(0, CH)]

        def blk_body(b, carry):
            pltpu.sync_copy(fsrc_hbm.at[s, pl.ds(b * FBLK, FBLK)], src_v)
            pltpu.sync_copy(fdst_hbm.at[s, pl.ds(b * FBLK, FBLK)], dst_v)
            pltpu.async_copy(x_hbm.at[src_v.at[0]], rows_a, sem_a)

            def pair_body(t, inner):
                pltpu.make_async_copy(dummy, rows_a, sem_a).wait()
                pltpu.async_copy(x_hbm.at[src_v.at[2 * t + 1]], rows_b, sem_b)
                pltpu.sync_copy(rows_a, acc.at[dst_v.at[2 * t]], add=True)
                pltpu.make_async_copy(dummy, rows_b, sem_b).wait()

                @pl.when(t < FBLK // 2 - 1)
                def _():
                    pltpu.async_copy(x_hbm.at[src_v.at[2 * t + 2]], rows_a,
                                     sem_a)

                pltpu.sync_copy(rows_b, acc.at[dst_v.at[2 * t + 1]], add=True)
                return inner

            return lax.fori_loop(0, FBLK // 2, pair_body, carry)

        lax.fori_loop(0, nblk, blk_body, jnp.int32(0))

    @pl.when(c == 0)
    def _():
        run(xlo_hbm)

    @pl.when(c == 1)
    def _():
        run(xhi_hbm)

    plsc.subcore_barrier()

    def writeout(a_hbm):
        @pl.when(s < 15)
        def _():
            pltpu.sync_copy(acc.at[pl.ds(s * _OUT_SL, _OUT_SL)],
                            a_hbm.at[pl.ds(s * _OUT_SL, _OUT_SL)])

        @pl.when(s == 15)
        def _():
            pltpu.sync_copy(acc.at[pl.ds(15 * _OUT_SL, _OUT_SL_LAST)],
                            a_hbm.at[pl.ds(15 * _OUT_SL, _OUT_SL_LAST)])

    @pl.when(c == 0)
    def _():
        writeout(alo_hbm)

    @pl.when(c == 1)
    def _():
        writeout(ahi_hbm)


def _seg_sum_f_sc(xlo, xhi, fsrc, fdst, cnts, zeros):
    return pl.kernel(
        _seg_sum_f_kernel,
        out_type=(jax.ShapeDtypeStruct((N, D), jnp.float32),
                  jax.ShapeDtypeStruct((N, D), jnp.float32)),
        mesh=_sc_mesh(),
        compiler_params=pltpu.CompilerParams(needs_layout_passes=False),
        scratch_types=[
            pltpu.VMEM((FBLK, CH), jnp.int32),
            pltpu.VMEM((FBLK, CH), jnp.int32),
            pltpu.VMEM((CH, D), jnp.float32),
            pltpu.VMEM((CH, D), jnp.float32),
            pltpu.VMEM((16,), jnp.int32),
            pltpu.VMEM_SHARED((ACC_ROWS, D), jnp.float32),
            pltpu.SemaphoreType.DMA,
            pltpu.SemaphoreType.DMA,
        ],
    )(xlo, xhi, fsrc.reshape(NS, FCAP_CH, CH), fdst.reshape(NS, FCAP_CH, CH),
      cnts, zeros)


def _emb_kernel(x_ref, w_ref, b_ref, lo_ref, hi_ref):
    h = jnp.dot(x_ref[...], w_ref[...], preferred_element_type=jnp.float32)
    h = jnp.maximum(h + b_ref[...], 0.0)
    lo_ref[...] = h[:, :D]
    hi_ref[...] = h[:, D:]


def _emb(x, w, b):
    return pl.pallas_call(
        _emb_kernel,
        out_shape=(jax.ShapeDtypeStruct((N, D), jnp.float32),
                   jax.ShapeDtypeStruct((N, D), jnp.float32)),
    )(x, w, b)


def _masked_bn(h, m, k, g, b):
    mu = jnp.sum(h * m, axis=0, keepdims=True) * (1.0 / k)
    d = h - mu
    var = jnp.sum(d * d * m, axis=0, keepdims=True) * (1.0 / k)
    return d * lax.rsqrt(var + BNEPS) * g + b


def _layer_ab_kernel(k_prev,
                     xlo_ref, xhi_ref, alo_ref, ahi_ref, m_ref,
                     w1_ref, b1_ref, g1_ref, bb1_ref,
                     w2_ref, b2_ref, g2_ref, bb2_ref, pw_ref,
                     ylo_ref, yhi_ref, sc_ref):
    m = m_ref[...]
    h = (jnp.dot(xlo_ref[...] + alo_ref[...], w1_ref[:D, :],
                 preferred_element_type=jnp.float32)
         + jnp.dot(xhi_ref[...] + ahi_ref[...], w1_ref[D:, :],
                   preferred_element_type=jnp.float32)
         + b1_ref[...])
    h = jnp.maximum(_masked_bn(h, m, k_prev, g1_ref[...], bb1_ref[...]), 0.0)
    h = (jnp.dot(h, w2_ref[...], preferred_element_type=jnp.float32)
         + b2_ref[...])
    y = _masked_bn(h, m, k_prev, g2_ref[...], bb2_ref[...])
    y = jnp.where(y > 0, y, 0.1 * y)

    pw = pw_ref[...]
    nrm = jnp.sqrt(jnp.sum(pw * pw)) + 1e-16
    score = jnp.tanh(jnp.dot(y, pw, preferred_element_type=jnp.float32) / nrm)
    ylo_ref[...] = y[:, :D]
    yhi_ref[...] = y[:, D:]
    sc_ref[...] = score


def _layer_c_kernel(k_new,
                    ylo_ref, yhi_ref, sc_ref, scr_ref, mr_ref, mc_ref,
                    xnlo_ref, xnhi_ref, mn_ref, read_ref):
    score_r = scr_ref[...]
    m_r = mr_ref[...]
    sm = jnp.where(m_r > 0, score_r, -2.0)

    u = lax.bitcast_convert_type(sm, jnp.uint32)
    ukey = jnp.where((u >> 31) != 0, ~u, u | jnp.uint32(0x80000000))

    def t_body(i, prefix):
        cand = prefix | (jnp.uint32(1) << (31 - i).astype(jnp.uint32))
        cnt = jnp.sum((ukey >= cand).astype(jnp.int32))
        return jnp.where(cnt >= k_new, cand, prefix)

    tkey = lax.fori_loop(0, 32, t_body, jnp.uint32(0))
    c_gt = jnp.sum((ukey > tkey).astype(jnp.int32))
    mrem = k_new - c_gt

    def r_body(i, prefix):
        bit = jnp.int32(1) << (13 - i).astype(jnp.int32)
        cap = prefix | (bit - 1)
        idx = lax.broadcasted_iota(jnp.int32, (1, N), 1)
        cnt = jnp.sum(((ukey == tkey) & (idx <= cap)).astype(jnp.int32))
        return jnp.where(cnt >= mrem, prefix, prefix | bit)

    ridx = lax.fori_loop(0, 14, r_body, jnp.int32(0))

    score = sc_ref[...]
    mc = lax.bitcast_convert_type(jnp.where(mc_ref[...] > 0, score, -2.0),
                                  jnp.uint32)
    ukc = jnp.where((mc >> 31) != 0, ~mc, mc | jnp.uint32(0x80000000))
    idc = lax.broadcasted_iota(jnp.int32, (N, 1), 0)
    sel = (ukc > tkey) | ((ukc == tkey) & (idc <= ridx) & (mrem > 0))
    mn = sel.astype(jnp.float32)
    mn_ref[...] = mn
    sc = score * mn
    xnlo = ylo_ref[...] * sc
    xnhi = yhi_ref[...] * sc
    xnlo_ref[...] = xnlo
    xnhi_ref[...] = xnhi
    mxlo = jnp.max(jnp.where(sel, xnlo, NEG_HUGE), axis=0, keepdims=True)
    mxhi = jnp.max(jnp.where(sel, xnhi, NEG_HUGE), axis=0, keepdims=True)
    mnlo = jnp.sum(xnlo, axis=0, keepdims=True) * (1.0 / k_new)
    mnhi = jnp.sum(xnhi, axis=0, keepdims=True) * (1.0 / k_new)
    read_ref[...] = jnp.concatenate([mxlo, mxhi, mnlo, mnhi], axis=1)


def _layer(k_prev, k_new, xlo, xhi, alo, ahi, mask, cp, g, b, pw):
    ylo, yhi, score = pl.pallas_call(
        functools.partial(_layer_ab_kernel, k_prev),
        out_shape=(jax.ShapeDtypeStruct((N, D), jnp.float32),
                   jax.ShapeDtypeStruct((N, D), jnp.float32),
                   jax.ShapeDtypeStruct((N, 1), jnp.float32)),
        compiler_params=pltpu.CompilerParams(
            vmem_limit_bytes=64 * 1024 * 1024),
    )(xlo, xhi, alo, ahi, mask,
      cp['lin1_W'], cp['lin1_b'].reshape(1, H), cp['bn_g'].reshape(1, H),
      cp['bn_b'].reshape(1, H),
      cp['lin2_W'], cp['lin2_b'].reshape(1, H),
      g.reshape(1, H), b.reshape(1, H), pw.reshape(H, 1))
    score_row = score.reshape(1, N)
    mask_row = mask.reshape(1, N)
    return pl.pallas_call(
        functools.partial(_layer_c_kernel, k_new),
        out_shape=(jax.ShapeDtypeStruct((N, D), jnp.float32),
                   jax.ShapeDtypeStruct((N, D), jnp.float32),
                   jax.ShapeDtypeStruct((N, 1), jnp.float32),
                   jax.ShapeDtypeStruct((1, 2 * H), jnp.float32)),
    )(ylo, yhi, score, score_row, mask_row, mask)


def _final_kernel(r1_ref, r2_ref, r3_ref, w_ref, b_ref, out_ref):
    def lk(v):
        return jnp.where(v > 0, v, 0.1 * v)

    agg = lk(r1_ref[...]) + lk(r2_ref[...]) + lk(r3_ref[...])
    out_ref[...] = (jnp.dot(agg, w_ref[...], preferred_element_type=jnp.float32)
                    + b_ref[...])


def _final(r1, r2, r3, w, b):
    return pl.pallas_call(
        _final_kernel,
        out_shape=jax.ShapeDtypeStruct((1, H), jnp.float32),
    )(r1, r2, r3, w, b.reshape(1, H))


def kernel(x, edge_index, batch, params):
    p = params
    src = edge_index[0].astype(jnp.int32)
    dst = edge_index[1].astype(jnp.int32)
    src2d = jnp.concatenate(
        [src, (jnp.arange(E_PAD, dtype=jnp.int32) * 613) & 8191]
    ).reshape(NS, CHUNKS * CH)
    dst2d = jnp.concatenate(
        [dst, N + (jnp.arange(E_PAD, dtype=jnp.int32) % 64)]
    ).reshape(NS, CHUNKS * CH)
    srcp = src2d.reshape(NS, CHUNKS, CH)
    dstp = dst2d.reshape(NS, CHUNKS, CH)
    zeros = jnp.zeros((ACC_ROWS, D), jnp.float32)

    xlo, xhi = _emb(x, p['emb_W'], p['emb_b'].reshape(1, H))
    mask = jnp.ones((N, 1), jnp.float32)
    k = N
    reads = []
    for i in (1, 2, 3):
        if i == 1:
            alo, ahi = _seg_sum_sc(xlo, xhi, srcp, dstp, zeros)
        else:
            fsrc, fdst, cnts = _edge_filter(src2d, dst2d, mask.reshape(N))
            alo, ahi = _seg_sum_f_sc(xlo, xhi, fsrc, fdst, cnts, zeros)
        k_new = int(math.ceil(0.5 * k))
        xlo, xhi, mask, read = _layer(
            k, k_new, xlo, xhi, alo, ahi, mask, p['conv%d' % i],
            p['bn%d_g' % i], p['bn%d_b' % i], p['pool%d_w' % i])
        k = k_new
        reads.append(read)
    return _final(reads[0], reads[1], reads[2], p['lin1_W'], p['lin1_b'])
